# baseline (speedup 1.0000x reference)
import functools

import jax
import jax.numpy as jnp
from jax import lax
from jax.experimental import pallas as pl
from jax.experimental.pallas import tpu as pltpu
from jax.experimental.pallas import tpu_sc as plsc

STATE_SIZE = 100
NUM_CLASSES = 1000
OUT_FEATURES = 128
BATCH = 4096

NUM_CORES = 2
NUM_SUBCORES = 16
NUM_WORKERS = NUM_CORES * NUM_SUBCORES
BAGS_PER_WORKER = BATCH // NUM_WORKERS
LANE = 16
NVEC = OUT_FEATURES // LANE
RING = 4


@functools.partial(
    pl.kernel,
    out_type=jax.ShapeDtypeStruct((BATCH, OUT_FEATURES), jnp.float32),
    mesh=plsc.VectorSubcoreMesh(
        core_axis_name="c", subcore_axis_name="s",
        num_cores=NUM_CORES, num_subcores=NUM_SUBCORES,
    ),
    scratch_types=[
        pltpu.VMEM((BAGS_PER_WORKER, STATE_SIZE), jnp.int32),
        pltpu.VMEM((OUT_FEATURES,), jnp.float32),
        pltpu.VMEM((BAGS_PER_WORKER, OUT_FEATURES), jnp.float32),
    ] + [pltpu.VMEM((STATE_SIZE, OUT_FEATURES), jnp.float32)
         for _ in range(RING)]
      + [pltpu.SemaphoreType.DMA] * RING,
)
def _embag(tok_hbm, w_hbm, b_hbm, out_hbm,
           tok, bvec, outb, r0, r1, r2, r3, g0, g1, g2, g3):
    rows = [r0, r1, r2, r3]
    gsem = [g0, g1, g2, g3]

    cid = lax.axis_index("c")
    sid = lax.axis_index("s")
    wid = cid * NUM_SUBCORES + sid

    def gather(j, b):
        pltpu.async_copy(w_hbm.at[tok.at[j]], rows[b], gsem[b])

    def gather_wait(j, b):
        pltpu.make_async_copy(w_hbm.at[tok.at[j]], rows[b], gsem[b]).wait()

    pltpu.sync_copy(tok_hbm.at[wid], tok)
    pltpu.sync_copy(b_hbm, bvec)
    bias_v = [bvec[pl.ds(k * LANE, LANE)] for k in range(NVEC)]

    for b in range(RING):
        gather(b, b)

    def _reduce(j, b):
        buf = rows[b]

        def body(r, acc):
            r2 = r * 2
            return tuple(acc[k] + (buf[r2, pl.ds(k * LANE, LANE)] +
                                   buf[r2 + 1, pl.ds(k * LANE, LANE)])
                         for k in range(NVEC))

        acc = lax.fori_loop(0, STATE_SIZE // 2, body, tuple(bias_v))
        for k in range(NVEC):
            outb[j, pl.ds(k * LANE, LANE)] = acc[k]

    def _lap(it, _):
        j0 = it * RING
        for b in range(RING):
            j = j0 + b
            gather_wait(j, b)
            _reduce(j, b)
            gather(j + RING, b)
        return 0

    lax.fori_loop(0, BAGS_PER_WORKER // RING - 1, _lap, 0)

    t0 = BAGS_PER_WORKER - RING
    for b in range(RING):
        gather_wait(t0 + b, b)
        _reduce(t0 + b, b)

    pltpu.sync_copy(outb, out_hbm.at[pl.ds(wid * BAGS_PER_WORKER,
                                           BAGS_PER_WORKER)])


def kernel(indices, weight, bias):
    offsets = jnp.arange(STATE_SIZE, dtype=indices.dtype) * NUM_CLASSES
    tokens = (indices + offsets[None, :]).astype(jnp.int32)
    tokens = tokens.reshape(NUM_WORKERS, BAGS_PER_WORKER, STATE_SIZE)
    return _embag(tokens, weight, bias.astype(jnp.float32))


# speedup vs baseline: 1.0070x; 1.0070x over previous
import functools

import jax
import jax.numpy as jnp
from jax import lax
from jax.experimental import pallas as pl
from jax.experimental.pallas import tpu as pltpu
from jax.experimental.pallas import tpu_sc as plsc

STATE_SIZE = 100
NUM_CLASSES = 1000
OUT_FEATURES = 128
BATCH = 4096

NUM_CORES = 2
NUM_SUBCORES = 16
NUM_WORKERS = NUM_CORES * NUM_SUBCORES
BAGS_PER_WORKER = BATCH // NUM_WORKERS
LANE = 16
NVEC = OUT_FEATURES // LANE
RING = 4


@functools.partial(
    pl.kernel,
    out_type=jax.ShapeDtypeStruct((BATCH, OUT_FEATURES), jnp.float32),
    mesh=plsc.VectorSubcoreMesh(
        core_axis_name="c", subcore_axis_name="s",
        num_cores=NUM_CORES, num_subcores=NUM_SUBCORES,
    ),
    scratch_types=[
        pltpu.VMEM((BAGS_PER_WORKER, STATE_SIZE), jnp.int32),
        pltpu.VMEM((OUT_FEATURES,), jnp.float32),
        pltpu.VMEM((BAGS_PER_WORKER, OUT_FEATURES), jnp.float32),
    ] + [pltpu.VMEM((STATE_SIZE, OUT_FEATURES), jnp.float32)
         for _ in range(RING)]
      + [pltpu.SemaphoreType.DMA] * (2 * RING),
)
def _embag(tok_hbm, w_hbm, b_hbm, out_hbm,
           tok, bvec, outb, r0, r1, r2, r3,
           g0, g1, g2, g3, h0, h1, h2, h3):
    rows = [r0, r1, r2, r3]
    gsem = [g0, g1, g2, g3]
    hsem = [h0, h1, h2, h3]
    SPLIT = 64

    cid = lax.axis_index("c")
    sid = lax.axis_index("s")
    wid = cid * NUM_SUBCORES + sid

    def gather(j, b):
        pltpu.async_copy(w_hbm.at[tok.at[j, pl.ds(0, SPLIT)]],
                         rows[b].at[pl.ds(0, SPLIT)], gsem[b])
        pltpu.async_copy(w_hbm.at[tok.at[j, pl.ds(SPLIT, STATE_SIZE - SPLIT)]],
                         rows[b].at[pl.ds(SPLIT, STATE_SIZE - SPLIT)], hsem[b])

    def gather_wait(j, b):
        pltpu.make_async_copy(w_hbm.at[tok.at[j, pl.ds(0, SPLIT)]],
                              rows[b].at[pl.ds(0, SPLIT)], gsem[b]).wait()
        pltpu.make_async_copy(w_hbm.at[tok.at[j, pl.ds(SPLIT, STATE_SIZE - SPLIT)]],
                              rows[b].at[pl.ds(SPLIT, STATE_SIZE - SPLIT)],
                              hsem[b]).wait()

    pltpu.sync_copy(tok_hbm.at[wid], tok)
    pltpu.sync_copy(b_hbm, bvec)
    bias_v = [bvec[pl.ds(k * LANE, LANE)] for k in range(NVEC)]

    for b in range(RING):
        gather(b, b)

    def _reduce(j, b):
        buf = rows[b]

        def body(r, acc):
            r2 = r * 2
            return tuple(acc[k] + (buf[r2, pl.ds(k * LANE, LANE)] +
                                   buf[r2 + 1, pl.ds(k * LANE, LANE)])
                         for k in range(NVEC))

        acc = lax.fori_loop(0, STATE_SIZE // 2, body, tuple(bias_v))
        for k in range(NVEC):
            outb[j, pl.ds(k * LANE, LANE)] = acc[k]

    def _lap(it, _):
        j0 = it * RING
        for b in range(RING):
            j = j0 + b
            gather_wait(j, b)
            _reduce(j, b)
            gather(j + RING, b)
        return 0

    lax.fori_loop(0, BAGS_PER_WORKER // RING - 1, _lap, 0)

    t0 = BAGS_PER_WORKER - RING
    for b in range(RING):
        gather_wait(t0 + b, b)
        _reduce(t0 + b, b)

    pltpu.sync_copy(outb, out_hbm.at[pl.ds(wid * BAGS_PER_WORKER,
                                           BAGS_PER_WORKER)])


def kernel(indices, weight, bias):
    offsets = jnp.arange(STATE_SIZE, dtype=indices.dtype) * NUM_CLASSES
    tokens = (indices + offsets[None, :]).astype(jnp.int32)
    tokens = tokens.reshape(NUM_WORKERS, BAGS_PER_WORKER, STATE_SIZE)
    return _embag(tokens, weight, bias.astype(jnp.float32))
